# Initial kernel scaffold; baseline (speedup 1.0000x reference)
import jax
import jax.numpy as jnp
from jax.experimental import pallas as pl


def kernel(x, emb, W1, b1, W2, b2, W3, b3):
    raise NotImplementedError("write your pallas kernel here")


# speedup vs baseline: 21.9859x; 21.9859x over previous
import functools

import jax
import jax.numpy as jnp
from jax import lax
from jax.experimental import pallas as pl
from jax.experimental.pallas import tpu as pltpu
from jax.experimental.pallas import tpu_sc as plsc

B = 16384
P = 64
V = 832
VP = 896
H1, H2 = 512, 256

NC, NS = 2, 16
NW = NC * NS
RW = B // NW
GRP = 16
NGRP = RW // GRP

BLK = 1024


_mesh = plsc.VectorSubcoreMesh(core_axis_name="c", subcore_axis_name="s")


@functools.partial(
    pl.kernel,
    mesh=_mesh,
    compiler_params=pltpu.CompilerParams(use_tc_tiling_on_sc=False,
                                         needs_layout_passes=False),
    out_type=jax.ShapeDtypeStruct((B, VP), jnp.float32),
    scratch_types=[
        pltpu.VMEM((P, RW), jnp.int32),
        pltpu.VMEM((GRP, VP), jnp.float32),
    ],
)
def _hist(xT_hbm, out_hbm, xv, buf):
    wid = lax.axis_index("s") * NC + lax.axis_index("c")
    base = wid * RW
    pltpu.sync_copy(xT_hbm.at[:, pl.ds(base, RW)], xv)

    lanes = lax.broadcasted_iota(jnp.int32, (16,), 0)
    ones = jnp.ones((16,), jnp.float32)
    zeros16 = jnp.zeros((16,), jnp.float32)

    def grp_body(g, carry):
        def zrow(r, c):
            def zcol(i, c2):
                buf[r, pl.ds(i * 16, 16)] = zeros16
                return c2
            return lax.fori_loop(0, VP // 16, zcol, c)
        lax.fori_loop(0, GRP, zrow, carry)

        def pbody(p, c):
            col = xv[p, pl.ds(g * GRP, GRP)]
            plsc.addupdate_scatter(buf, [lanes, col], ones)
            return c
        lax.fori_loop(0, P, pbody, carry)

        pltpu.sync_copy(buf, out_hbm.at[pl.ds(base + g * GRP, GRP), :])
        return carry

    lax.fori_loop(0, NGRP, grp_body, 0)


def _e1_body(emb_ref, w1_ref, out_ref):
    out_ref[...] = lax.dot_general(
        emb_ref[...], w1_ref[...], (((1,), (1,)), ((), ())),
        preferred_element_type=jnp.float32)


_e1_call = pl.pallas_call(
    _e1_body,
    out_shape=jax.ShapeDtypeStruct((VP, H1), jnp.float32),
)


def _mlp_body(c_ref, e1_ref, b1_ref, w2_ref, b2_ref, w3_ref, b3_ref, o_ref):
    acc = jnp.dot(c_ref[...], e1_ref[...], preferred_element_type=jnp.float32)
    h1 = jnp.maximum(acc + b1_ref[...], 0.0)
    h2 = lax.dot_general(h1, w2_ref[...], (((1,), (1,)), ((), ())),
                         preferred_element_type=jnp.float32)
    h2 = jnp.maximum(h2 + b2_ref[...], 0.0)
    o_ref[...] = jnp.sum(h2 * w3_ref[...], axis=1, keepdims=True) + b3_ref[...]


_mlp_call = pl.pallas_call(
    _mlp_body,
    grid=(B // BLK,),
    in_specs=[
        pl.BlockSpec((BLK, VP), lambda i: (i, 0)),
        pl.BlockSpec((VP, H1), lambda i: (0, 0)),
        pl.BlockSpec((1, H1), lambda i: (0, 0)),
        pl.BlockSpec((H2, H1), lambda i: (0, 0)),
        pl.BlockSpec((1, H2), lambda i: (0, 0)),
        pl.BlockSpec((1, H2), lambda i: (0, 0)),
        pl.BlockSpec((1, 1), lambda i: (0, 0)),
    ],
    out_specs=pl.BlockSpec((BLK, 1), lambda i: (i, 0)),
    out_shape=jax.ShapeDtypeStruct((B, 1), jnp.float32),
)


def kernel(x, emb, W1, b1, W2, b2, W3, b3):
    xT = x.astype(jnp.int32).T
    emb_p = jnp.pad(emb, ((0, VP - V), (0, 0)))
    C = _hist(xT)
    E1 = _e1_call(emb_p, W1)
    return _mlp_call(C, E1, b1.reshape(1, H1), W2, b2.reshape(1, H2),
                     W3, b3.reshape(1, 1))


# speedup vs baseline: 29.2084x; 1.3285x over previous
import functools

import jax
import jax.numpy as jnp
from jax import lax
from jax.experimental import pallas as pl
from jax.experimental.pallas import tpu as pltpu
from jax.experimental.pallas import tpu_sc as plsc

B = 16384
P = 64
V = 832
VP = 896
H1, H2 = 512, 256

NC, NS = 2, 16
NW = NC * NS
RW = B // NW
GRP = 16
NGRP = RW // GRP

BLK = 1024


_mesh = plsc.VectorSubcoreMesh(core_axis_name="c", subcore_axis_name="s")


NBUF = 2


@functools.partial(
    pl.kernel,
    mesh=_mesh,
    compiler_params=pltpu.CompilerParams(use_tc_tiling_on_sc=False,
                                         needs_layout_passes=False),
    out_type=jax.ShapeDtypeStruct((B, VP), jnp.float32),
    scratch_types=[
        pltpu.VMEM((RW * P,), jnp.int32),
        pltpu.VMEM((GRP, VP), jnp.float32),
        pltpu.VMEM((GRP, VP), jnp.float32),
        pltpu.SemaphoreType.DMA,
        pltpu.SemaphoreType.DMA,
    ],
)
def _hist(x_hbm, out_hbm, xv, buf0, buf1, sem0, sem1):
    wid = lax.axis_index("s") * NC + lax.axis_index("c")
    base = wid * RW
    pltpu.sync_copy(x_hbm.at[pl.ds(base * P, RW * P)], xv)

    lanes = lax.broadcasted_iota(jnp.int32, (16,), 0)
    lanesP = lanes * P
    ones = jnp.ones((16,), jnp.float32)
    zeros16 = jnp.zeros((16,), jnp.float32)
    bufs = (buf0, buf1)
    sems = (sem0, sem1)

    def pair_body(gp, carry):
        for b in range(NBUF):
            g = gp * NBUF + b
            buf, sem = bufs[b], sems[b]

            @pl.when(gp > 0)
            def _wait():
                pltpu.make_async_copy(
                    out_hbm.at[pl.ds(0, GRP), :], buf, sem).wait()

            for r in range(GRP):
                for i in range(VP // 16):
                    buf[r, pl.ds(i * 16, 16)] = zeros16

            gbase = g * GRP * P
            for p in range(P):
                col = plsc.load_gather(xv, [lanesP + (gbase + p)])
                plsc.addupdate_scatter(buf, [lanes, col], ones)

            pltpu.async_copy(buf, out_hbm.at[pl.ds(base + g * GRP, GRP), :],
                             sem)
        return carry

    lax.fori_loop(0, NGRP // NBUF, pair_body, 0)

    for b in range(NBUF):
        pltpu.make_async_copy(
            out_hbm.at[pl.ds(0, GRP), :], bufs[b], sems[b]).wait()


def _e1_body(emb_ref, w1_ref, out_ref):
    out_ref[...] = lax.dot_general(
        emb_ref[...], w1_ref[...], (((1,), (1,)), ((), ())),
        preferred_element_type=jnp.float32)


_e1_call = pl.pallas_call(
    _e1_body,
    out_shape=jax.ShapeDtypeStruct((VP, H1), jnp.float32),
)


def _mlp_body(c_ref, e1_ref, b1_ref, w2_ref, b2_ref, w3_ref, b3_ref, o_ref):
    acc = jnp.dot(c_ref[...], e1_ref[...], preferred_element_type=jnp.float32)
    h1 = jnp.maximum(acc + b1_ref[...], 0.0)
    h2 = lax.dot_general(h1, w2_ref[...], (((1,), (1,)), ((), ())),
                         preferred_element_type=jnp.float32)
    h2 = jnp.maximum(h2 + b2_ref[...], 0.0)
    o_ref[...] = jnp.sum(h2 * w3_ref[...], axis=1, keepdims=True) + b3_ref[...]


_mlp_call = pl.pallas_call(
    _mlp_body,
    grid=(B // BLK,),
    in_specs=[
        pl.BlockSpec((BLK, VP), lambda i: (i, 0)),
        pl.BlockSpec((VP, H1), lambda i: (0, 0)),
        pl.BlockSpec((1, H1), lambda i: (0, 0)),
        pl.BlockSpec((H2, H1), lambda i: (0, 0)),
        pl.BlockSpec((1, H2), lambda i: (0, 0)),
        pl.BlockSpec((1, H2), lambda i: (0, 0)),
        pl.BlockSpec((1, 1), lambda i: (0, 0)),
    ],
    out_specs=pl.BlockSpec((BLK, 1), lambda i: (i, 0)),
    out_shape=jax.ShapeDtypeStruct((B, 1), jnp.float32),
)


def kernel(x, emb, W1, b1, W2, b2, W3, b3):
    x_flat = x.astype(jnp.int32).reshape(-1)
    emb_p = jnp.pad(emb, ((0, VP - V), (0, 0)))
    C = _hist(x_flat)
    E1 = _e1_call(emb_p, W1)
    return _mlp_call(C, E1, b1.reshape(1, H1), W2, b2.reshape(1, H2),
                     W3, b3.reshape(1, 1))


# speedup vs baseline: 38.2040x; 1.3080x over previous
import functools

import jax
import jax.numpy as jnp
from jax import lax
from jax.experimental import pallas as pl
from jax.experimental.pallas import tpu as pltpu
from jax.experimental.pallas import tpu_sc as plsc

B = 16384
P = 64
V = 832
VP = 896
H1, H2 = 512, 256

NC, NS = 2, 16
NW = NC * NS
RW = B // NW
GRP = 16
NGRP = RW // GRP

BLK = 1024


_mesh = plsc.VectorSubcoreMesh(core_axis_name="c", subcore_axis_name="s")


NBUF = 2


@functools.partial(
    pl.kernel,
    mesh=_mesh,
    compiler_params=pltpu.CompilerParams(use_tc_tiling_on_sc=False,
                                         needs_layout_passes=False),
    out_type=jax.ShapeDtypeStruct((B * VP,), jnp.float32),
    scratch_types=[
        pltpu.VMEM((RW * P,), jnp.int32),
        pltpu.VMEM((GRP * VP,), jnp.float32),
        pltpu.VMEM((GRP * VP,), jnp.float32),
        pltpu.SemaphoreType.DMA,
        pltpu.SemaphoreType.DMA,
    ],
)
def _hist(x_hbm, out_hbm, xv, buf0, buf1, sem0, sem1):
    wid = lax.axis_index("s") * NC + lax.axis_index("c")
    base = wid * RW
    pltpu.sync_copy(x_hbm.at[pl.ds(base * P, RW * P)], xv)

    lanes = lax.broadcasted_iota(jnp.int32, (16,), 0)
    lanesP = lanes * P
    lane_base = (lanes >> 3) * (VP * 8) + (lanes & 7) * 128
    ones = jnp.ones((16,), jnp.float32)
    zeros16 = jnp.zeros((16,), jnp.float32)
    bufs = (buf0, buf1)
    sems = (sem0, sem1)

    def pair_body(gp, carry):
        for b in range(NBUF):
            g = gp * NBUF + b
            buf, sem = bufs[b], sems[b]

            @pl.when(gp > 0)
            def _wait():
                pltpu.make_async_copy(
                    out_hbm.at[pl.ds(0, GRP * VP)], buf, sem).wait()

            for i in range(GRP * VP // 16):
                buf[pl.ds(i * 16, 16)] = zeros16

            gbase = g * GRP * P
            for p in range(P):
                col = plsc.load_gather(xv, [lanesP + (gbase + p)])
                off = lane_base + ((col >> 7) << 10) + (col & 127)
                plsc.addupdate_scatter(buf, [off], ones)

            pltpu.async_copy(
                buf, out_hbm.at[pl.ds((base + g * GRP) * VP, GRP * VP)], sem)
        return carry

    lax.fori_loop(0, NGRP // NBUF, pair_body, 0)

    for b in range(NBUF):
        pltpu.make_async_copy(
            out_hbm.at[pl.ds(0, GRP * VP)], bufs[b], sems[b]).wait()


def _e1_body(emb_ref, w1_ref, out_ref):
    out_ref[...] = lax.dot_general(
        emb_ref[...], w1_ref[...], (((1,), (1,)), ((), ())),
        preferred_element_type=jnp.float32)


_e1_call = pl.pallas_call(
    _e1_body,
    out_shape=jax.ShapeDtypeStruct((VP, H1), jnp.float32),
)


def _mlp_body(c_ref, e1_ref, b1_ref, w2_ref, b2_ref, w3_ref, b3_ref, o_ref):
    c4 = c_ref[...]
    c = jnp.concatenate(
        [c4[:, j].reshape(BLK, 128) for j in range(VP // 128)], axis=1)
    acc = jnp.dot(c, e1_ref[...], preferred_element_type=jnp.float32)
    h1 = jnp.maximum(acc + b1_ref[...], 0.0)
    h2 = lax.dot_general(h1, w2_ref[...], (((1,), (1,)), ((), ())),
                         preferred_element_type=jnp.float32)
    h2 = jnp.maximum(h2 + b2_ref[...], 0.0)
    o_ref[...] = jnp.sum(h2 * w3_ref[...], axis=1, keepdims=True) + b3_ref[...]


_mlp_call = pl.pallas_call(
    _mlp_body,
    grid=(B // BLK,),
    in_specs=[
        pl.BlockSpec((BLK // 8, VP // 128, 8, 128), lambda i: (i, 0, 0, 0)),
        pl.BlockSpec((VP, H1), lambda i: (0, 0)),
        pl.BlockSpec((1, H1), lambda i: (0, 0)),
        pl.BlockSpec((H2, H1), lambda i: (0, 0)),
        pl.BlockSpec((1, H2), lambda i: (0, 0)),
        pl.BlockSpec((1, H2), lambda i: (0, 0)),
        pl.BlockSpec((1, 1), lambda i: (0, 0)),
    ],
    out_specs=pl.BlockSpec((BLK, 1), lambda i: (i, 0)),
    out_shape=jax.ShapeDtypeStruct((B, 1), jnp.float32),
)


def kernel(x, emb, W1, b1, W2, b2, W3, b3):
    x_flat = x.astype(jnp.int32).reshape(-1)
    emb_p = jnp.pad(emb, ((0, VP - V), (0, 0)))
    C = _hist(x_flat).reshape(B // 8, VP // 128, 8, 128)

    E1 = _e1_call(emb_p, W1)
    return _mlp_call(C, E1, b1.reshape(1, H1), W2, b2.reshape(1, H2),
                     W3, b3.reshape(1, 1))


# speedup vs baseline: 54.2104x; 1.4190x over previous
import functools

import jax
import jax.numpy as jnp
import numpy as np
from jax import lax
from jax.experimental import pallas as pl
from jax.experimental.pallas import tpu as pltpu
from jax.experimental.pallas import tpu_sc as plsc

B = 16384
P = 64
V = 832
VP = 1024
VPW = VP // 4
T = VPW // 128
H1, H2 = 512, 256

NC, NS = 2, 16
NW = NC * NS
RW = B // NW
GRP = 16
NGRP = RW // GRP
NBUF = 2

BLK = 1024

_PERM = np.concatenate([
    4 * (128 * j + np.arange(128)) + k for j in range(T) for k in range(4)
])

_mesh = plsc.VectorSubcoreMesh(core_axis_name="c", subcore_axis_name="s")


@functools.partial(
    pl.kernel,
    mesh=_mesh,
    compiler_params=pltpu.CompilerParams(use_tc_tiling_on_sc=False,
                                         needs_layout_passes=False),
    out_type=jax.ShapeDtypeStruct((B * VPW,), jnp.int32),
    scratch_types=[
        pltpu.VMEM((RW * P,), jnp.int32),
        pltpu.VMEM((GRP * VPW,), jnp.int32),
        pltpu.VMEM((GRP * VPW,), jnp.int32),
        pltpu.SemaphoreType.DMA,
        pltpu.SemaphoreType.DMA,
    ],
)
def _hist(x_hbm, out_hbm, xv, buf0, buf1, sem0, sem1):
    wid = lax.axis_index("s") * NC + lax.axis_index("c")
    base = wid * RW
    pltpu.sync_copy(x_hbm.at[pl.ds(base * P, RW * P)], xv)

    lanes = lax.broadcasted_iota(jnp.int32, (16,), 0)
    lanesP = lanes * P
    lane_base = (lanes >> 3) * (VPW * 8) + (lanes & 7) * 128
    one = jnp.ones((16,), jnp.int32)
    zeros16 = jnp.zeros((16,), jnp.int32)
    bufs = (buf0, buf1)
    sems = (sem0, sem1)

    def pair_body(gp, carry):
        for bi in range(NBUF):
            g = gp * NBUF + bi
            buf, sem = bufs[bi], sems[bi]

            @pl.when(gp > 0)
            def _wait():
                pltpu.make_async_copy(
                    out_hbm.at[pl.ds(0, GRP * VPW)], buf, sem).wait()

            for i in range(GRP * VPW // 16):
                buf[pl.ds(i * 16, 16)] = zeros16

            gbase = g * GRP * P
            for p in range(P):
                col = plsc.load_gather(
                    xv, [lanesP + gbase + ((lanes + p) & (P - 1))])
                w = col >> 2
                off = lane_base + ((w >> 7) << 10) + (w & 127)
                val = one << ((col & 3) << 3)
                plsc.addupdate_scatter(buf, [off], val)

            pltpu.async_copy(
                buf, out_hbm.at[pl.ds((base + g * GRP) * VPW, GRP * VPW)],
                sem)
        return carry

    lax.fori_loop(0, NGRP // NBUF, pair_body, 0)

    for bi in range(NBUF):
        pltpu.make_async_copy(
            out_hbm.at[pl.ds(0, GRP * VPW)], bufs[bi], sems[bi]).wait()


def _e1_body(emb_ref, w1_ref, out_ref):
    out_ref[...] = lax.dot_general(
        emb_ref[...], w1_ref[...], (((1,), (1,)), ((), ())),
        preferred_element_type=jnp.float32)


_e1_call = pl.pallas_call(
    _e1_body,
    out_shape=jax.ShapeDtypeStruct((VP, H1), jnp.float32),
)


def _mlp_body(c_ref, e1_ref, b1_ref, w2_ref, b2_ref, w3_ref, b3_ref, o_ref):
    c4 = c_ref[...]
    pieces = []
    for j in range(T):
        wj = c4[:, j].reshape(BLK, 128)
        for k in range(4):
            pieces.append(((wj >> (8 * k)) & 0xFF).astype(jnp.float32))
    c = jnp.concatenate(pieces, axis=1)
    acc = jnp.dot(c, e1_ref[...], preferred_element_type=jnp.float32)
    h1 = jnp.maximum(acc + b1_ref[...], 0.0)
    h2 = lax.dot_general(h1, w2_ref[...], (((1,), (1,)), ((), ())),
                         preferred_element_type=jnp.float32)
    h2 = jnp.maximum(h2 + b2_ref[...], 0.0)
    o_ref[...] = jnp.sum(h2 * w3_ref[...], axis=1, keepdims=True) + b3_ref[...]


_mlp_call = pl.pallas_call(
    _mlp_body,
    grid=(B // BLK,),
    in_specs=[
        pl.BlockSpec((BLK // 8, T, 8, 128), lambda i: (i, 0, 0, 0)),
        pl.BlockSpec((VP, H1), lambda i: (0, 0)),
        pl.BlockSpec((1, H1), lambda i: (0, 0)),
        pl.BlockSpec((H2, H1), lambda i: (0, 0)),
        pl.BlockSpec((1, H2), lambda i: (0, 0)),
        pl.BlockSpec((1, H2), lambda i: (0, 0)),
        pl.BlockSpec((1, 1), lambda i: (0, 0)),
    ],
    out_specs=pl.BlockSpec((BLK, 1), lambda i: (i, 0)),
    out_shape=jax.ShapeDtypeStruct((B, 1), jnp.float32),
)


def kernel(x, emb, W1, b1, W2, b2, W3, b3):
    x_flat = x.astype(jnp.int32).reshape(-1)
    emb_p = jnp.pad(emb, ((0, VP - V), (0, 0)))
    emb_perm = emb_p[_PERM]
    Cp = _hist(x_flat).reshape(B // 8, T, 8, 128)
    E1 = _e1_call(emb_perm, W1)
    return _mlp_call(Cp, E1, b1.reshape(1, H1), W2, b2.reshape(1, H2),
                     W3, b3.reshape(1, 1))


# speedup vs baseline: 74.3740x; 1.3720x over previous
import functools

import jax
import jax.numpy as jnp
import numpy as np
from jax import lax
from jax.experimental import pallas as pl
from jax.experimental.pallas import tpu as pltpu
from jax.experimental.pallas import tpu_sc as plsc

B = 16384
P = 64
V = 832
VP = 1024
VPW = VP // 4
T = VPW // 128
H1, H2 = 512, 256

NC, NS = 2, 16
NW = NC * NS
RW = B // NW
GRP = 16
NGRP = RW // GRP
NBUF = 2

BLK = 1024

_PERM = np.concatenate([
    4 * (128 * j + np.arange(128)) + k for j in range(T) for k in range(4)
])

_mesh = plsc.VectorSubcoreMesh(core_axis_name="c", subcore_axis_name="s")


@functools.partial(
    pl.kernel,
    mesh=_mesh,
    compiler_params=pltpu.CompilerParams(use_tc_tiling_on_sc=False,
                                         needs_layout_passes=False),
    out_type=jax.ShapeDtypeStruct((B * VPW,), jnp.int32),
    scratch_types=[
        pltpu.VMEM((RW * P,), jnp.int32),
        pltpu.VMEM((GRP * VPW,), jnp.int32),
        pltpu.VMEM((GRP * VPW,), jnp.int32),
        pltpu.SemaphoreType.DMA,
        pltpu.SemaphoreType.DMA,
    ],
)
def _hist(x_hbm, out_hbm, xv, buf0, buf1, sem0, sem1):
    wid = lax.axis_index("s") * NC + lax.axis_index("c")
    base = wid * RW
    pltpu.sync_copy(x_hbm.at[pl.ds(base * P, RW * P)], xv)

    lanes = lax.broadcasted_iota(jnp.int32, (16,), 0)
    lanesP = lanes * P
    lane_base = (lanes >> 3) * (VPW * 8) + (lanes & 7) * 128
    one = jnp.ones((16,), jnp.int32)
    zeros16 = jnp.zeros((16,), jnp.int32)
    bufs = (buf0, buf1)
    sems = (sem0, sem1)

    def pair_body(gp, carry):
        for bi in range(NBUF):
            g = gp * NBUF + bi
            buf, sem = bufs[bi], sems[bi]

            @pl.when(gp > 0)
            def _wait():
                pltpu.make_async_copy(
                    out_hbm.at[pl.ds(0, GRP * VPW)], buf, sem).wait()

            @plsc.parallel_loop(0, GRP * VPW, 16, unroll=8)
            def _zero(i):
                buf[pl.ds(i, 16)] = zeros16

            gbase = g * GRP * P

            @plsc.parallel_loop(0, P, 1, unroll=8)
            def _scat(p):
                col = plsc.load_gather(
                    xv, [lanesP + gbase + ((lanes + p) & (P - 1))])
                w = col >> 2
                off = lane_base + ((w >> 7) << 10) + (w & 127)
                val = one << ((col & 3) << 3)
                plsc.addupdate_scatter(buf, [off], val)

            pltpu.async_copy(
                buf, out_hbm.at[pl.ds((base + g * GRP) * VPW, GRP * VPW)],
                sem)
        return carry

    lax.fori_loop(0, NGRP // NBUF, pair_body, 0)

    for bi in range(NBUF):
        pltpu.make_async_copy(
            out_hbm.at[pl.ds(0, GRP * VPW)], bufs[bi], sems[bi]).wait()


def _e1_body(emb_ref, w1_ref, out_ref):
    out_ref[...] = lax.dot_general(
        emb_ref[...], w1_ref[...], (((1,), (1,)), ((), ())),
        preferred_element_type=jnp.float32)


_e1_call = pl.pallas_call(
    _e1_body,
    out_shape=jax.ShapeDtypeStruct((VP, H1), jnp.float32),
)


def _mlp_body(c_ref, e1_ref, b1_ref, w2_ref, b2_ref, w3_ref, b3_ref, o_ref):
    c4 = c_ref[...]
    pieces = []
    for j in range(T):
        wj = c4[:, j].reshape(BLK, 128)
        for k in range(4):
            pieces.append(((wj >> (8 * k)) & 0xFF).astype(jnp.float32))
    c = jnp.concatenate(pieces, axis=1)
    acc = jnp.dot(c, e1_ref[...], preferred_element_type=jnp.float32)
    h1 = jnp.maximum(acc + b1_ref[...], 0.0)
    h2 = lax.dot_general(h1, w2_ref[...], (((1,), (1,)), ((), ())),
                         preferred_element_type=jnp.float32)
    h2 = jnp.maximum(h2 + b2_ref[...], 0.0)
    o_ref[...] = jnp.sum(h2 * w3_ref[...], axis=1, keepdims=True) + b3_ref[...]


_mlp_call = pl.pallas_call(
    _mlp_body,
    grid=(B // BLK,),
    in_specs=[
        pl.BlockSpec((BLK // 8, T, 8, 128), lambda i: (i, 0, 0, 0)),
        pl.BlockSpec((VP, H1), lambda i: (0, 0)),
        pl.BlockSpec((1, H1), lambda i: (0, 0)),
        pl.BlockSpec((H2, H1), lambda i: (0, 0)),
        pl.BlockSpec((1, H2), lambda i: (0, 0)),
        pl.BlockSpec((1, H2), lambda i: (0, 0)),
        pl.BlockSpec((1, 1), lambda i: (0, 0)),
    ],
    out_specs=pl.BlockSpec((BLK, 1), lambda i: (i, 0)),
    out_shape=jax.ShapeDtypeStruct((B, 1), jnp.float32),
)


def kernel(x, emb, W1, b1, W2, b2, W3, b3):
    x_flat = x.astype(jnp.int32).reshape(-1)
    emb_p = jnp.pad(emb, ((0, VP - V), (0, 0)))
    emb_perm = emb_p[_PERM]
    Cp = _hist(x_flat).reshape(B // 8, T, 8, 128)
    E1 = _e1_call(emb_perm, W1)
    return _mlp_call(Cp, E1, b1.reshape(1, H1), W2, b2.reshape(1, H2),
                     W3, b3.reshape(1, 1))


# speedup vs baseline: 74.9135x; 1.0073x over previous
import functools

import jax
import jax.numpy as jnp
import numpy as np
from jax import lax
from jax.experimental import pallas as pl
from jax.experimental.pallas import tpu as pltpu
from jax.experimental.pallas import tpu_sc as plsc

B = 16384
P = 64
V = 832
VP = 1024
VPW = VP // 4
T = VPW // 128
H1, H2 = 512, 256

NC, NS = 2, 16
NW = NC * NS
RW = B // NW
GRP = 16
NGRP = RW // GRP
NBUF = 2

BLK = 1024

_PERM = np.concatenate([
    4 * (128 * j + np.arange(128)) + k for j in range(T) for k in range(4)
])

_mesh = plsc.VectorSubcoreMesh(core_axis_name="c", subcore_axis_name="s")


@functools.partial(
    pl.kernel,
    mesh=_mesh,
    compiler_params=pltpu.CompilerParams(use_tc_tiling_on_sc=False,
                                         needs_layout_passes=False),
    out_type=jax.ShapeDtypeStruct((B * VPW,), jnp.int32),
    scratch_types=[
        pltpu.VMEM((RW, P), jnp.int32),
        pltpu.VMEM((GRP * VPW,), jnp.int32),
        pltpu.VMEM((GRP * VPW,), jnp.int32),
        pltpu.SemaphoreType.DMA,
        pltpu.SemaphoreType.DMA,
    ],
)
def _hist(x_hbm, out_hbm, xv, buf0, buf1, sem0, sem1):
    wid = lax.axis_index("s") * NC + lax.axis_index("c")
    base = wid * RW
    pltpu.sync_copy(x_hbm.at[pl.ds(base, RW), :], xv)

    lanes = lax.broadcasted_iota(jnp.int32, (16,), 0)
    lane_base = (lanes >> 3) * (VPW * 8) + (lanes & 7) * 128
    one = jnp.ones((16,), jnp.int32)
    zeros16 = jnp.zeros((16,), jnp.int32)
    bufs = (buf0, buf1)
    sems = (sem0, sem1)

    def pair_body(gp, carry):
        for bi in range(NBUF):
            g = gp * NBUF + bi
            buf, sem = bufs[bi], sems[bi]

            @pl.when(gp > 0)
            def _wait():
                pltpu.make_async_copy(
                    out_hbm.at[pl.ds(0, GRP * VPW)], buf, sem).wait()

            @plsc.parallel_loop(0, GRP * VPW, 16, unroll=8)
            def _zero(i):
                buf[pl.ds(i, 16)] = zeros16

            grow = g * GRP

            @plsc.parallel_loop(0, P, 1, unroll=8)
            def _scat(p):
                col = plsc.load_gather(
                    xv, [grow + lanes, (lanes + p) & (P - 1)])
                w = col >> 2
                off = lane_base + ((w >> 7) << 10) + (w & 127)
                val = one << ((col & 3) << 3)
                plsc.addupdate_scatter(buf, [off], val)

            pltpu.async_copy(
                buf, out_hbm.at[pl.ds((base + g * GRP) * VPW, GRP * VPW)],
                sem)
        return carry

    lax.fori_loop(0, NGRP // NBUF, pair_body, 0)

    for bi in range(NBUF):
        pltpu.make_async_copy(
            out_hbm.at[pl.ds(0, GRP * VPW)], bufs[bi], sems[bi]).wait()


def _e1_body(emb_ref, w1_ref, out_ref):
    out_ref[...] = lax.dot_general(
        emb_ref[...], w1_ref[...], (((1,), (1,)), ((), ())),
        preferred_element_type=jnp.float32).astype(jnp.bfloat16)


_e1_call = pl.pallas_call(
    _e1_body,
    out_shape=jax.ShapeDtypeStruct((VP, H1), jnp.bfloat16),
)


def _mlp_body(c_ref, e1_ref, b1_ref, w2_ref, b2_ref, w3_ref, b3_ref, o_ref):
    c4 = c_ref[...]
    pieces = []
    for j in range(T):
        wj = c4[:, j].reshape(BLK, 128)
        for k in range(4):
            pieces.append(((wj >> (8 * k)) & 0xFF).astype(jnp.bfloat16))
    c = jnp.concatenate(pieces, axis=1)
    acc = jnp.dot(c, e1_ref[...], preferred_element_type=jnp.float32)
    h1 = jnp.maximum(acc + b1_ref[...], 0.0)
    h2 = lax.dot_general(h1, w2_ref[...], (((1,), (1,)), ((), ())),
                         preferred_element_type=jnp.float32)
    h2 = jnp.maximum(h2 + b2_ref[...], 0.0)
    o_ref[...] = jnp.sum(h2 * w3_ref[...], axis=1, keepdims=True) + b3_ref[...]


_mlp_call = pl.pallas_call(
    _mlp_body,
    grid=(B // BLK,),
    in_specs=[
        pl.BlockSpec((BLK // 8, T, 8, 128), lambda i: (i, 0, 0, 0)),
        pl.BlockSpec((VP, H1), lambda i: (0, 0)),
        pl.BlockSpec((1, H1), lambda i: (0, 0)),
        pl.BlockSpec((H2, H1), lambda i: (0, 0)),
        pl.BlockSpec((1, H2), lambda i: (0, 0)),
        pl.BlockSpec((1, H2), lambda i: (0, 0)),
        pl.BlockSpec((1, 1), lambda i: (0, 0)),
    ],
    out_specs=pl.BlockSpec((BLK, 1), lambda i: (i, 0)),
    out_shape=jax.ShapeDtypeStruct((B, 1), jnp.float32),
)


def kernel(x, emb, W1, b1, W2, b2, W3, b3):
    emb_p = jnp.pad(emb, ((0, VP - V), (0, 0)))
    emb_perm = emb_p[_PERM]
    Cp = _hist(x.astype(jnp.int32)).reshape(B // 8, T, 8, 128)
    E1 = _e1_call(emb_perm, W1)
    return _mlp_call(Cp, E1, b1.reshape(1, H1), W2, b2.reshape(1, H2),
                     W3, b3.reshape(1, 1))


# speedup vs baseline: 77.4273x; 1.0336x over previous
import functools

import jax
import jax.numpy as jnp
import numpy as np
from jax import lax
from jax.experimental import pallas as pl
from jax.experimental.pallas import tpu as pltpu
from jax.experimental.pallas import tpu_sc as plsc

B = 16384
P = 64
V = 832
VP = 1024
VPW = VP // 4
T = VPW // 128
H1, H2 = 512, 256

NC, NS = 2, 16
NW = NC * NS
RW = B // NW
GRP = 16
NGRP = RW // GRP
NBUF = 2

BLK = 1024


_mesh = plsc.VectorSubcoreMesh(core_axis_name="c", subcore_axis_name="s")


@functools.partial(
    pl.kernel,
    mesh=_mesh,
    compiler_params=pltpu.CompilerParams(use_tc_tiling_on_sc=False,
                                         needs_layout_passes=False),
    out_type=jax.ShapeDtypeStruct((B * VPW,), jnp.int32),
    scratch_types=[
        pltpu.VMEM((RW, P), jnp.int32),
        pltpu.VMEM((GRP * VPW,), jnp.int32),
        pltpu.VMEM((GRP * VPW,), jnp.int32),
        pltpu.SemaphoreType.DMA,
        pltpu.SemaphoreType.DMA,
    ],
)
def _hist(x_hbm, out_hbm, xv, buf0, buf1, sem0, sem1):
    wid = lax.axis_index("s") * NC + lax.axis_index("c")
    base = wid * RW
    pltpu.sync_copy(x_hbm.at[pl.ds(base, RW), :], xv)

    lanes = lax.broadcasted_iota(jnp.int32, (16,), 0)
    lane_base = (lanes >> 3) * (VPW * 8) + (lanes & 7) * 128
    one = jnp.ones((16,), jnp.int32)
    zeros16 = jnp.zeros((16,), jnp.int32)
    bufs = (buf0, buf1)
    sems = (sem0, sem1)

    def pair_body(gp, carry):
        for bi in range(NBUF):
            g = gp * NBUF + bi
            buf, sem = bufs[bi], sems[bi]

            @pl.when(gp > 0)
            def _wait():
                pltpu.make_async_copy(
                    out_hbm.at[pl.ds(0, GRP * VPW)], buf, sem).wait()

            @plsc.parallel_loop(0, GRP * VPW, 16, unroll=8)
            def _zero(i):
                buf[pl.ds(i, 16)] = zeros16

            grow = g * GRP

            @plsc.parallel_loop(0, P, 1, unroll=8)
            def _scat(p):
                col = plsc.load_gather(
                    xv, [grow + lanes, (lanes + p) & (P - 1)])
                off = lane_base + ((col >> 9) << 10) + (col & 127)
                val = one << (((col >> 7) & 3) << 3)
                plsc.addupdate_scatter(buf, [off], val)

            pltpu.async_copy(
                buf, out_hbm.at[pl.ds((base + g * GRP) * VPW, GRP * VPW)],
                sem)
        return carry

    lax.fori_loop(0, NGRP // NBUF, pair_body, 0)

    for bi in range(NBUF):
        pltpu.make_async_copy(
            out_hbm.at[pl.ds(0, GRP * VPW)], bufs[bi], sems[bi]).wait()


def _e1_body(emb_ref, w1_ref, out_ref):
    out_ref[V:, :] = jnp.zeros((VP - V, H1), jnp.bfloat16)
    out_ref[:V, :] = lax.dot_general(
        emb_ref[...], w1_ref[...], (((1,), (1,)), ((), ())),
        preferred_element_type=jnp.float32).astype(jnp.bfloat16)


_e1_call = pl.pallas_call(
    _e1_body,
    out_shape=jax.ShapeDtypeStruct((VP, H1), jnp.bfloat16),
)


def _mlp_body(c_ref, e1_ref, b1_ref, w2_ref, b2_ref, w3_ref, b3_ref, o_ref):
    c4 = c_ref[...]
    pieces = []
    for j in range(T):
        wj = c4[:, j].reshape(BLK, 128)
        for k in range(4):
            pieces.append(((wj >> (8 * k)) & 0xFF).astype(jnp.bfloat16))
    c = jnp.concatenate(pieces, axis=1)
    acc = jnp.dot(c, e1_ref[...], preferred_element_type=jnp.float32)
    h1 = jnp.maximum(acc + b1_ref[...], 0.0)
    h2 = lax.dot_general(h1, w2_ref[...], (((1,), (1,)), ((), ())),
                         preferred_element_type=jnp.float32)
    h2 = jnp.maximum(h2 + b2_ref[...], 0.0)
    o_ref[...] = jnp.sum(h2 * w3_ref[...], axis=1, keepdims=True) + b3_ref[...]


_mlp_call = pl.pallas_call(
    _mlp_body,
    grid=(B // BLK,),
    in_specs=[
        pl.BlockSpec((BLK // 8, T, 8, 128), lambda i: (i, 0, 0, 0)),
        pl.BlockSpec((VP, H1), lambda i: (0, 0)),
        pl.BlockSpec((1, H1), lambda i: (0, 0)),
        pl.BlockSpec((H2, H1), lambda i: (0, 0)),
        pl.BlockSpec((1, H2), lambda i: (0, 0)),
        pl.BlockSpec((1, H2), lambda i: (0, 0)),
        pl.BlockSpec((1, 1), lambda i: (0, 0)),
    ],
    out_specs=pl.BlockSpec((BLK, 1), lambda i: (i, 0)),
    out_shape=jax.ShapeDtypeStruct((B, 1), jnp.float32),
)


def kernel(x, emb, W1, b1, W2, b2, W3, b3):
    Cp = _hist(x.astype(jnp.int32)).reshape(B // 8, T, 8, 128)
    E1 = _e1_call(emb, W1)
    return _mlp_call(Cp, E1, b1.reshape(1, H1), W2, b2.reshape(1, H2),
                     W3, b3.reshape(1, 1))
